# baseline (speedup 1.0000x reference)
import functools

import jax
import jax.numpy as jnp
from jax import lax
from jax.experimental import pallas as pl
from jax.experimental.pallas import tpu as pltpu
from jax.experimental.pallas import tpu_sc as plsc

N = 10000
E = 320000
D = 128
H1 = 64
H2 = 32

NCORES = 2
NSUB = 16
NW = NCORES * NSUB
EB = 128
NBUF = 8
NBATCH = -(-(-(-E // (NW * EB))) // NBUF) * NBUF
E_PAD = NW * EB * NBATCH
NPT = 632
N_PAD = NSUB * NPT
DEGW = 16

ROWBLK = 400
GRID = N // ROWBLK


def _sc_mesh():
    return plsc.VectorSubcoreMesh(core_axis_name="c", subcore_axis_name="s")


def _sc_degree(dst3, ones_row, zeros16):
    @functools.partial(
        pl.kernel,
        out_type=jax.ShapeDtypeStruct((NCORES, N_PAD, DEGW), jnp.float32),
        mesh=_sc_mesh(),
        compiler_params=pltpu.CompilerParams(use_tc_tiling_on_sc=False),
        scratch_types=[
            pltpu.VMEM((NBATCH, EB), jnp.int32),
            pltpu.VMEM((EB, DEGW), jnp.float32),
            pltpu.VMEM_SHARED((N_PAD, DEGW), jnp.float32),
        ] + [pltpu.SemaphoreType.DMA] * NBUF,
    )
    def k(dst3_hbm, ones_hbm, zero_hbm, out_hbm, dst_v, ones_v, acc_sh, *sems):
        c = lax.axis_index("c")
        s = lax.axis_index("s")
        wid = s * NCORES + c
        pltpu.sync_copy(dst3_hbm.at[wid], dst_v)
        pltpu.sync_copy(ones_hbm, ones_v)
        pltpu.sync_copy(zero_hbm.at[pl.ds(s * NPT, NPT)],
                        acc_sh.at[pl.ds(s * NPT, NPT)])
        plsc.subcore_barrier()

        def body(k_, carry):
            for i in range(NBUF):
                j = NBUF * k_ + i

                @pl.when(k_ > 0)
                def _():
                    pltpu.make_async_copy(
                        ones_v, acc_sh.at[dst_v.at[j - NBUF]], sems[i]).wait()

                pltpu.async_copy(ones_v, acc_sh.at[dst_v.at[j]], sems[i],
                                 add=True)
            return carry

        lax.fori_loop(0, NBATCH // NBUF, body, 0)
        for i in range(NBUF):
            j = NBATCH - NBUF + i
            pltpu.make_async_copy(ones_v, acc_sh.at[dst_v.at[j]], sems[i]).wait()
        plsc.subcore_barrier()
        pltpu.sync_copy(acc_sh.at[pl.ds(s * NPT, NPT)],
                        out_hbm.at[c, pl.ds(s * NPT, NPT)])

    return k(dst3, ones_row, zeros16)


def _sc_conv(hws, src3, dst3, zeros, feat):
    @functools.partial(
        pl.kernel,
        out_type=jax.ShapeDtypeStruct((NCORES, N_PAD, feat), jnp.float32),
        mesh=_sc_mesh(),
        compiler_params=pltpu.CompilerParams(use_tc_tiling_on_sc=False),
        scratch_types=[
            pltpu.VMEM((NBATCH, EB), jnp.int32),
            pltpu.VMEM((NBATCH, EB), jnp.int32),
            pltpu.VMEM_SHARED((N_PAD, feat), jnp.float32),
        ] + [pltpu.VMEM((EB, feat), jnp.float32)] * NBUF
          + [pltpu.SemaphoreType.DMA] * (2 * NBUF),
    )
    def k(hws_hbm, src3_hbm, dst3_hbm, zero_hbm, out_hbm, src_v, dst_v, acc_sh,
          *rest):
        bufs = rest[:NBUF]
        gsem = rest[NBUF:2 * NBUF]
        ssem = rest[2 * NBUF:]
        c = lax.axis_index("c")
        s = lax.axis_index("s")
        wid = s * NCORES + c
        pltpu.sync_copy(src3_hbm.at[wid], src_v)
        pltpu.sync_copy(dst3_hbm.at[wid], dst_v)
        pltpu.sync_copy(zero_hbm.at[pl.ds(s * NPT, NPT)],
                        acc_sh.at[pl.ds(s * NPT, NPT)])
        plsc.subcore_barrier()

        for i in range(NBUF):
            pltpu.async_copy(hws_hbm.at[src_v.at[i]], bufs[i], gsem[i])

        def body(k_, carry):
            for i in range(NBUF):
                j = NBUF * k_ + i

                @pl.when(k_ > 0)
                def _():
                    pltpu.make_async_copy(
                        bufs[i], acc_sh.at[dst_v.at[j - NBUF]], ssem[i]).wait()
                    pltpu.async_copy(
                        hws_hbm.at[src_v.at[j]], bufs[i], gsem[i])

            for i in range(NBUF):
                j = NBUF * k_ + i
                pltpu.make_async_copy(
                    hws_hbm.at[src_v.at[j]], bufs[i], gsem[i]).wait()
                pltpu.async_copy(bufs[i], acc_sh.at[dst_v.at[j]], ssem[i],
                                 add=True)
            return carry

        lax.fori_loop(0, NBATCH // NBUF, body, 0)
        for i in range(NBUF):
            j = NBATCH - NBUF + i
            pltpu.make_async_copy(
                bufs[i], acc_sh.at[dst_v.at[j]], ssem[i]).wait()
        plsc.subcore_barrier()
        pltpu.sync_copy(acc_sh.at[pl.ds(s * NPT, NPT)],
                        out_hbm.at[c, pl.ds(s * NPT, NPT)])

    return k(hws, src3, dst3, zeros)


def _leaky(x):
    return jnp.where(x >= 0, x, 0.01 * x)


def _tc_pre(x, dcnt, W1, b1, W2, b2, Wc1):
    def body(x_ref, d_ref, W1_ref, b1_ref, W2_ref, b2_ref, Wc1_ref, hws_ref, dis_ref):
        h = jnp.dot(x_ref[...], W1_ref[...], preferred_element_type=jnp.float32)
        h = _leaky(h + b1_ref[...])
        h = jnp.dot(h, W2_ref[...], preferred_element_type=jnp.float32)
        h = _leaky(h + b2_ref[...])
        hw = jnp.dot(h, Wc1_ref[...], preferred_element_type=jnp.float32)
        deg = d_ref[0, :, 0:1] + d_ref[1, :, 0:1] + 1.0
        dis = 1.0 / jnp.sqrt(deg)
        hws_ref[...] = hw * dis
        dis_ref[...] = dis

    return pl.pallas_call(
        body,
        grid=(GRID,),
        in_specs=[
            pl.BlockSpec((ROWBLK, D), lambda i: (i, 0)),
            pl.BlockSpec((NCORES, ROWBLK, DEGW), lambda i: (0, i, 0)),
            pl.BlockSpec((D, 256), lambda i: (0, 0)),
            pl.BlockSpec((1, 256), lambda i: (0, 0)),
            pl.BlockSpec((256, D), lambda i: (0, 0)),
            pl.BlockSpec((1, D), lambda i: (0, 0)),
            pl.BlockSpec((D, H1), lambda i: (0, 0)),
        ],
        out_specs=[
            pl.BlockSpec((ROWBLK, H1), lambda i: (i, 0)),
            pl.BlockSpec((ROWBLK, 1), lambda i: (i, 0)),
        ],
        out_shape=[
            jax.ShapeDtypeStruct((N, H1), jnp.float32),
            jax.ShapeDtypeStruct((N, 1), jnp.float32),
        ],
    )(x, dcnt, W1, b1, W2, b2, Wc1)


def _tc_mid(acc, hws1, dis, b1, Wc2):
    def body(acc_ref, hws_ref, dis_ref, b1_ref, Wc2_ref, out_ref):
        dis = dis_ref[...]
        agg = acc_ref[0] + acc_ref[1] + hws_ref[...]
        t = _leaky(agg * dis + b1_ref[...])
        t = t / jnp.sqrt(jnp.sum(t * t, axis=1, keepdims=True))
        out_ref[...] = jnp.dot(t, Wc2_ref[...], preferred_element_type=jnp.float32) * dis

    return pl.pallas_call(
        body,
        grid=(GRID,),
        in_specs=[
            pl.BlockSpec((NCORES, ROWBLK, H1), lambda i: (0, i, 0)),
            pl.BlockSpec((ROWBLK, H1), lambda i: (i, 0)),
            pl.BlockSpec((ROWBLK, 1), lambda i: (i, 0)),
            pl.BlockSpec((1, H1), lambda i: (0, 0)),
            pl.BlockSpec((H1, H2), lambda i: (0, 0)),
        ],
        out_specs=pl.BlockSpec((ROWBLK, H2), lambda i: (i, 0)),
        out_shape=jax.ShapeDtypeStruct((N, H2), jnp.float32),
    )(acc, hws1, dis, b1, Wc2)


def _tc_post(acc, hws2, dis, b2, Wp, bp):
    def body(acc_ref, hws_ref, dis_ref, b2_ref, Wp_ref, bp_ref, out_ref):
        dis = dis_ref[...]
        agg = acc_ref[0] + acc_ref[1] + hws_ref[...]
        t = _leaky(agg * dis + b2_ref[...])
        t = t / jnp.sqrt(jnp.sum(t * t, axis=1, keepdims=True))
        out_ref[...] = jnp.dot(t, Wp_ref[...], preferred_element_type=jnp.float32) + bp_ref[...]

    return pl.pallas_call(
        body,
        grid=(GRID,),
        in_specs=[
            pl.BlockSpec((NCORES, ROWBLK, H2), lambda i: (0, i, 0)),
            pl.BlockSpec((ROWBLK, H2), lambda i: (i, 0)),
            pl.BlockSpec((ROWBLK, 1), lambda i: (i, 0)),
            pl.BlockSpec((1, H2), lambda i: (0, 0)),
            pl.BlockSpec((H2, 2), lambda i: (0, 0)),
            pl.BlockSpec((1, 2), lambda i: (0, 0)),
        ],
        out_specs=pl.BlockSpec((ROWBLK, 2), lambda i: (i, 0)),
        out_shape=jax.ShapeDtypeStruct((N, 2), jnp.float32),
    )(acc, hws2, dis, b2, Wp, bp)


def kernel(x, edge_index, W_pre1, b_pre1, W_pre2, b_pre2, W_c1, b_c1, W_c2, b_c2,
           W_post, b_post):
    pad = E_PAD - E
    src3 = jnp.concatenate(
        [edge_index[0], jnp.zeros((pad,), jnp.int32)]).reshape(NW, NBATCH, EB)
    dst3 = jnp.concatenate(
        [edge_index[1], jnp.full((pad,), N, jnp.int32)]).reshape(NW, NBATCH, EB)

    ones_row = jnp.ones((EB, DEGW), jnp.float32)
    z16 = jnp.zeros((N_PAD, DEGW), jnp.float32)
    z64 = jnp.zeros((N_PAD, H1), jnp.float32)
    z32 = jnp.zeros((N_PAD, H2), jnp.float32)

    dcnt = _sc_degree(dst3, ones_row, z16)
    hws1, dis = _tc_pre(x, dcnt, W_pre1, b_pre1.reshape(1, -1),
                        W_pre2, b_pre2.reshape(1, -1), W_c1)
    acc1 = _sc_conv(hws1, src3, dst3, z64, H1)
    hws2 = _tc_mid(acc1, hws1, dis, b_c1.reshape(1, -1), W_c2)
    acc2 = _sc_conv(hws2, src3, dst3, z32, H2)
    out = _tc_post(acc2, hws2, dis, b_c2.reshape(1, -1), W_post, b_post.reshape(1, -1))
    return out


# speedup vs baseline: 1.8690x; 1.8690x over previous
import functools

import jax
import jax.numpy as jnp
from jax import lax
from jax.experimental import pallas as pl
from jax.experimental.pallas import tpu as pltpu
from jax.experimental.pallas import tpu_sc as plsc

N = 10000
E = 320000
D = 128
H1 = 64
H2 = 32

NCORES = 2
NSUB = 16
NW = NCORES * NSUB
EB = 128
NBUF = 3
NBATCH = -(-(-(-E // (NW * EB))) // NBUF) * NBUF
E_PAD = NW * EB * NBATCH
NPT = 632
N_PAD = NSUB * NPT
DEGW = 16

ROWBLK = 400
GRID = N // ROWBLK


def _sc_mesh():
    return plsc.VectorSubcoreMesh(core_axis_name="c", subcore_axis_name="s")


def _sc_degree(dst3, ones_row, zeros16):
    @functools.partial(
        pl.kernel,
        out_type=jax.ShapeDtypeStruct((NCORES, N_PAD, DEGW), jnp.float32),
        mesh=_sc_mesh(),
        compiler_params=pltpu.CompilerParams(use_tc_tiling_on_sc=False),
        scratch_types=[
            pltpu.VMEM((NBATCH, EB), jnp.int32),
            pltpu.VMEM((EB, DEGW), jnp.float32),
            pltpu.VMEM_SHARED((N_PAD, DEGW), jnp.float32),
        ] + [pltpu.SemaphoreType.DMA] * NBUF,
    )
    def k(dst3_hbm, ones_hbm, zero_hbm, out_hbm, dst_v, ones_v, acc_sh, *sems):
        c = lax.axis_index("c")
        s = lax.axis_index("s")
        wid = s * NCORES + c
        pltpu.sync_copy(dst3_hbm.at[wid], dst_v)
        pltpu.sync_copy(ones_hbm, ones_v)
        pltpu.sync_copy(zero_hbm.at[pl.ds(s * NPT, NPT)],
                        acc_sh.at[pl.ds(s * NPT, NPT)])
        plsc.subcore_barrier()

        def body(k_, carry):
            for i in range(NBUF):
                j = NBUF * k_ + i

                @pl.when(k_ > 0)
                def _():
                    pltpu.make_async_copy(
                        ones_v, acc_sh.at[dst_v.at[j - NBUF]], sems[i]).wait()

                pltpu.async_copy(ones_v, acc_sh.at[dst_v.at[j]], sems[i],
                                 add=True)
            return carry

        lax.fori_loop(0, NBATCH // NBUF, body, 0)
        for i in range(NBUF):
            j = NBATCH - NBUF + i
            pltpu.make_async_copy(ones_v, acc_sh.at[dst_v.at[j]], sems[i]).wait()
        plsc.subcore_barrier()
        pltpu.sync_copy(acc_sh.at[pl.ds(s * NPT, NPT)],
                        out_hbm.at[c, pl.ds(s * NPT, NPT)])

    return k(dst3, ones_row, zeros16)


def _sc_conv(hws, src3, dst3, zeros, feat):
    @functools.partial(
        pl.kernel,
        out_type=jax.ShapeDtypeStruct((NCORES, N_PAD, feat), jnp.float32),
        mesh=_sc_mesh(),
        compiler_params=pltpu.CompilerParams(use_tc_tiling_on_sc=False),
        scratch_types=[
            pltpu.VMEM((NBATCH, EB), jnp.int32),
            pltpu.VMEM((NBATCH, EB), jnp.int32),
            pltpu.VMEM_SHARED((N_PAD, feat), jnp.float32),
            pltpu.VMEM_SHARED((N_PAD, feat), jnp.float32),
        ] + [pltpu.VMEM((EB, feat), jnp.float32)] * NBUF
          + [pltpu.SemaphoreType.DMA] * NBUF,
    )
    def k(hws_hbm, src3_hbm, dst3_hbm, zero_hbm, out_hbm, src_v, dst_v, acc_sh,
          hws_sh, *rest):
        bufs = rest[:NBUF]
        gsem = rest[NBUF:]
        c = lax.axis_index("c")
        s = lax.axis_index("s")
        wid = s * NCORES + c
        pltpu.sync_copy(src3_hbm.at[wid], src_v)
        pltpu.sync_copy(dst3_hbm.at[wid], dst_v)
        pltpu.sync_copy(zero_hbm.at[pl.ds(s * NPT, NPT)],
                        acc_sh.at[pl.ds(s * NPT, NPT)])
        pltpu.sync_copy(hws_hbm.at[pl.ds(s * NPT, NPT)],
                        hws_sh.at[pl.ds(s * NPT, NPT)])
        plsc.subcore_barrier()

        for i in range(NBUF):
            pltpu.async_copy(hws_sh.at[src_v.at[i]], bufs[i], gsem[i])

        def body(k_, carry):
            for i in range(NBUF):
                j = NBUF * k_ + i
                pltpu.make_async_copy(
                    hws_sh.at[src_v.at[j]], bufs[i], gsem[i]).wait()
                pltpu.sync_copy(bufs[i], acc_sh.at[dst_v.at[j]], add=True)

                @pl.when(j + NBUF < NBATCH)
                def _():
                    pltpu.async_copy(
                        hws_sh.at[src_v.at[j + NBUF]], bufs[i], gsem[i])
            return carry

        lax.fori_loop(0, NBATCH // NBUF, body, 0)
        plsc.subcore_barrier()
        pltpu.sync_copy(acc_sh.at[pl.ds(s * NPT, NPT)],
                        out_hbm.at[c, pl.ds(s * NPT, NPT)])

    return k(hws, src3, dst3, zeros)


def _leaky(x):
    return jnp.where(x >= 0, x, 0.01 * x)


def _tc_pre(x, dcnt, W1, b1, W2, b2, Wc1):
    def body(x_ref, d_ref, W1_ref, b1_ref, W2_ref, b2_ref, Wc1_ref, hws_ref, dis_ref):
        h = jnp.dot(x_ref[...], W1_ref[...], preferred_element_type=jnp.float32)
        h = _leaky(h + b1_ref[...])
        h = jnp.dot(h, W2_ref[...], preferred_element_type=jnp.float32)
        h = _leaky(h + b2_ref[...])
        hw = jnp.dot(h, Wc1_ref[...], preferred_element_type=jnp.float32)
        deg = d_ref[0, :, 0:1] + d_ref[1, :, 0:1] + 1.0
        dis = 1.0 / jnp.sqrt(deg)
        hws_ref[...] = hw * dis
        dis_ref[...] = dis

    return pl.pallas_call(
        body,
        grid=(GRID,),
        in_specs=[
            pl.BlockSpec((ROWBLK, D), lambda i: (i, 0)),
            pl.BlockSpec((NCORES, ROWBLK, DEGW), lambda i: (0, i, 0)),
            pl.BlockSpec((D, 256), lambda i: (0, 0)),
            pl.BlockSpec((1, 256), lambda i: (0, 0)),
            pl.BlockSpec((256, D), lambda i: (0, 0)),
            pl.BlockSpec((1, D), lambda i: (0, 0)),
            pl.BlockSpec((D, H1), lambda i: (0, 0)),
        ],
        out_specs=[
            pl.BlockSpec((ROWBLK, H1), lambda i: (i, 0)),
            pl.BlockSpec((ROWBLK, 1), lambda i: (i, 0)),
        ],
        out_shape=[
            jax.ShapeDtypeStruct((N_PAD, H1), jnp.float32),
            jax.ShapeDtypeStruct((N, 1), jnp.float32),
        ],
    )(x, dcnt, W1, b1, W2, b2, Wc1)


def _tc_mid(acc, hws1, dis, b1, Wc2):
    def body(acc_ref, hws_ref, dis_ref, b1_ref, Wc2_ref, out_ref):
        dis = dis_ref[...]
        agg = acc_ref[0] + acc_ref[1] + hws_ref[...]
        t = _leaky(agg * dis + b1_ref[...])
        t = t / jnp.sqrt(jnp.sum(t * t, axis=1, keepdims=True))
        out_ref[...] = jnp.dot(t, Wc2_ref[...], preferred_element_type=jnp.float32) * dis

    return pl.pallas_call(
        body,
        grid=(GRID,),
        in_specs=[
            pl.BlockSpec((NCORES, ROWBLK, H1), lambda i: (0, i, 0)),
            pl.BlockSpec((ROWBLK, H1), lambda i: (i, 0)),
            pl.BlockSpec((ROWBLK, 1), lambda i: (i, 0)),
            pl.BlockSpec((1, H1), lambda i: (0, 0)),
            pl.BlockSpec((H1, H2), lambda i: (0, 0)),
        ],
        out_specs=pl.BlockSpec((ROWBLK, H2), lambda i: (i, 0)),
        out_shape=jax.ShapeDtypeStruct((N_PAD, H2), jnp.float32),
    )(acc, hws1, dis, b1, Wc2)


def _tc_post(acc, hws2, dis, b2, Wp, bp):
    def body(acc_ref, hws_ref, dis_ref, b2_ref, Wp_ref, bp_ref, out_ref):
        dis = dis_ref[...]
        agg = acc_ref[0] + acc_ref[1] + hws_ref[...]
        t = _leaky(agg * dis + b2_ref[...])
        t = t / jnp.sqrt(jnp.sum(t * t, axis=1, keepdims=True))
        out_ref[...] = jnp.dot(t, Wp_ref[...], preferred_element_type=jnp.float32) + bp_ref[...]

    return pl.pallas_call(
        body,
        grid=(GRID,),
        in_specs=[
            pl.BlockSpec((NCORES, ROWBLK, H2), lambda i: (0, i, 0)),
            pl.BlockSpec((ROWBLK, H2), lambda i: (i, 0)),
            pl.BlockSpec((ROWBLK, 1), lambda i: (i, 0)),
            pl.BlockSpec((1, H2), lambda i: (0, 0)),
            pl.BlockSpec((H2, 2), lambda i: (0, 0)),
            pl.BlockSpec((1, 2), lambda i: (0, 0)),
        ],
        out_specs=pl.BlockSpec((ROWBLK, 2), lambda i: (i, 0)),
        out_shape=jax.ShapeDtypeStruct((N, 2), jnp.float32),
    )(acc, hws2, dis, b2, Wp, bp)


def kernel(x, edge_index, W_pre1, b_pre1, W_pre2, b_pre2, W_c1, b_c1, W_c2, b_c2,
           W_post, b_post):
    pad = E_PAD - E
    src3 = jnp.concatenate(
        [edge_index[0], jnp.zeros((pad,), jnp.int32)]).reshape(NW, NBATCH, EB)
    dst3 = jnp.concatenate(
        [edge_index[1], jnp.full((pad,), N, jnp.int32)]).reshape(NW, NBATCH, EB)

    ones_row = jnp.ones((EB, DEGW), jnp.float32)
    z16 = jnp.zeros((N_PAD, DEGW), jnp.float32)
    z64 = jnp.zeros((N_PAD, H1), jnp.float32)
    z32 = jnp.zeros((N_PAD, H2), jnp.float32)

    dcnt = _sc_degree(dst3, ones_row, z16)
    hws1, dis = _tc_pre(x, dcnt, W_pre1, b_pre1.reshape(1, -1),
                        W_pre2, b_pre2.reshape(1, -1), W_c1)
    acc1 = _sc_conv(hws1, src3, dst3, z64, H1)
    hws2 = _tc_mid(acc1, hws1, dis, b_c1.reshape(1, -1), W_c2)
    acc2 = _sc_conv(hws2, src3, dst3, z32, H2)
    out = _tc_post(acc2, hws2, dis, b_c2.reshape(1, -1), W_post, b_post.reshape(1, -1))
    return out


# speedup vs baseline: 2.0683x; 1.1066x over previous
import functools

import jax
import jax.numpy as jnp
from jax import lax
from jax.experimental import pallas as pl
from jax.experimental.pallas import tpu as pltpu
from jax.experimental.pallas import tpu_sc as plsc

N = 10000
E = 320000
D = 128
H1 = 64
H2 = 32

NCORES = 2
NSUB = 16
NW = NCORES * NSUB
EPT = E // NW
EB = 80
NBATCH = EPT // EB
NBUF_DEG = 5
NBUF = 5
NPT = 632
N_PAD = NSUB * NPT
DEGW = 16

ROWBLK = 400
GRID = N // ROWBLK


def _sc_mesh():
    return plsc.VectorSubcoreMesh(core_axis_name="c", subcore_axis_name="s")


def _load_idx(er_hbm, row, wid, idx_v):
    pltpu.sync_copy(er_hbm.at[row, wid], idx_v)


def _sc_degree(er, ones_row, zeros16):
    @functools.partial(
        pl.kernel,
        out_type=jax.ShapeDtypeStruct((NCORES, N_PAD, DEGW), jnp.float32),
        mesh=_sc_mesh(),
        compiler_params=pltpu.CompilerParams(use_tc_tiling_on_sc=False),
        scratch_types=[
            pltpu.VMEM((NBATCH, EB), jnp.int32),
            pltpu.VMEM((EB, DEGW), jnp.float32),
            pltpu.VMEM_SHARED((N_PAD, DEGW), jnp.float32),
        ] + [pltpu.SemaphoreType.DMA] * NBUF_DEG,
    )
    def k(er_hbm, ones_hbm, zero_hbm, out_hbm, dst_v, ones_v, acc_sh, *sems):
        c = lax.axis_index("c")
        s = lax.axis_index("s")
        wid = s * NCORES + c
        _load_idx(er_hbm, 1, wid, dst_v)
        pltpu.sync_copy(ones_hbm, ones_v)
        pltpu.sync_copy(zero_hbm.at[pl.ds(s * NPT, NPT)],
                        acc_sh.at[pl.ds(s * NPT, NPT)])
        plsc.subcore_barrier()

        def body(k_, carry):
            for i in range(NBUF_DEG):
                j = NBUF_DEG * k_ + i

                @pl.when(k_ > 0)
                def _():
                    pltpu.make_async_copy(
                        ones_v, acc_sh.at[dst_v.at[j - NBUF_DEG]],
                        sems[i]).wait()

                pltpu.async_copy(ones_v, acc_sh.at[dst_v.at[j]], sems[i],
                                 add=True)
            return carry

        lax.fori_loop(0, NBATCH // NBUF_DEG, body, 0)
        for i in range(NBUF_DEG):
            j = NBATCH - NBUF_DEG + i
            pltpu.make_async_copy(ones_v, acc_sh.at[dst_v.at[j]], sems[i]).wait()
        plsc.subcore_barrier()
        pltpu.sync_copy(acc_sh.at[pl.ds(s * NPT, NPT)],
                        out_hbm.at[c, pl.ds(s * NPT, NPT)])

    return k(er, ones_row, zeros16)


def _sc_conv(hws, er, zeros, feat):
    @functools.partial(
        pl.kernel,
        out_type=jax.ShapeDtypeStruct((NCORES, N_PAD, feat), jnp.float32),
        mesh=_sc_mesh(),
        compiler_params=pltpu.CompilerParams(use_tc_tiling_on_sc=False),
        scratch_types=[
            pltpu.VMEM((NBATCH, EB), jnp.int32),
            pltpu.VMEM((NBATCH, EB), jnp.int32),
            pltpu.VMEM_SHARED((N_PAD, feat), jnp.float32),
            pltpu.VMEM_SHARED((N_PAD, feat), jnp.float32),
        ] + [pltpu.VMEM((EB, feat), jnp.float32)] * NBUF
          + [pltpu.SemaphoreType.DMA] * NBUF,
    )
    def k(hws_hbm, er_hbm, zero_hbm, out_hbm, src_v, dst_v, acc_sh, hws_sh,
          *rest):
        bufs = rest[:NBUF]
        gsem = rest[NBUF:]
        c = lax.axis_index("c")
        s = lax.axis_index("s")
        wid = s * NCORES + c
        _load_idx(er_hbm, 0, wid, src_v)
        _load_idx(er_hbm, 1, wid, dst_v)
        pltpu.sync_copy(zero_hbm.at[pl.ds(s * NPT, NPT)],
                        acc_sh.at[pl.ds(s * NPT, NPT)])
        pltpu.sync_copy(hws_hbm.at[pl.ds(s * NPT, NPT)],
                        hws_sh.at[pl.ds(s * NPT, NPT)])
        plsc.subcore_barrier()

        for i in range(NBUF):
            pltpu.async_copy(hws_sh.at[src_v.at[i]], bufs[i], gsem[i])

        def body(k_, carry):
            for i in range(NBUF):
                j = NBUF * k_ + i
                pltpu.make_async_copy(
                    hws_sh.at[src_v.at[j]], bufs[i], gsem[i]).wait()
                pltpu.sync_copy(bufs[i], acc_sh.at[dst_v.at[j]], add=True)

                @pl.when(j + NBUF < NBATCH)
                def _():
                    pltpu.async_copy(
                        hws_sh.at[src_v.at[j + NBUF]], bufs[i], gsem[i])
            return carry

        lax.fori_loop(0, NBATCH // NBUF, body, 0)
        plsc.subcore_barrier()
        pltpu.sync_copy(acc_sh.at[pl.ds(s * NPT, NPT)],
                        out_hbm.at[c, pl.ds(s * NPT, NPT)])

    return k(hws, er, zeros)


def _leaky(x):
    return jnp.where(x >= 0, x, 0.01 * x)


def _tc_pre(x, dcnt, W1, b1, W2, b2, Wc1):
    def body(x_ref, d_ref, W1_ref, b1_ref, W2_ref, b2_ref, Wc1_ref, hws_ref, dis_ref):
        h = jnp.dot(x_ref[...], W1_ref[...], preferred_element_type=jnp.float32)
        h = _leaky(h + b1_ref[...])
        h = jnp.dot(h, W2_ref[...], preferred_element_type=jnp.float32)
        h = _leaky(h + b2_ref[...])
        hw = jnp.dot(h, Wc1_ref[...], preferred_element_type=jnp.float32)
        deg = d_ref[0, :, 0:1] + d_ref[1, :, 0:1] + 1.0
        dis = 1.0 / jnp.sqrt(deg)
        hws_ref[...] = hw * dis
        dis_ref[...] = dis

    return pl.pallas_call(
        body,
        grid=(GRID,),
        in_specs=[
            pl.BlockSpec((ROWBLK, D), lambda i: (i, 0)),
            pl.BlockSpec((NCORES, ROWBLK, DEGW), lambda i: (0, i, 0)),
            pl.BlockSpec((D, 256), lambda i: (0, 0)),
            pl.BlockSpec((1, 256), lambda i: (0, 0)),
            pl.BlockSpec((256, D), lambda i: (0, 0)),
            pl.BlockSpec((1, D), lambda i: (0, 0)),
            pl.BlockSpec((D, H1), lambda i: (0, 0)),
        ],
        out_specs=[
            pl.BlockSpec((ROWBLK, H1), lambda i: (i, 0)),
            pl.BlockSpec((ROWBLK, 1), lambda i: (i, 0)),
        ],
        out_shape=[
            jax.ShapeDtypeStruct((N_PAD, H1), jnp.float32),
            jax.ShapeDtypeStruct((N, 1), jnp.float32),
        ],
    )(x, dcnt, W1, b1, W2, b2, Wc1)


def _tc_mid(acc, hws1, dis, b1, Wc2):
    def body(acc_ref, hws_ref, dis_ref, b1_ref, Wc2_ref, out_ref):
        dis = dis_ref[...]
        agg = acc_ref[0] + acc_ref[1] + hws_ref[...]
        t = _leaky(agg * dis + b1_ref[...])
        t = t / jnp.sqrt(jnp.sum(t * t, axis=1, keepdims=True))
        out_ref[...] = jnp.dot(t, Wc2_ref[...], preferred_element_type=jnp.float32) * dis

    return pl.pallas_call(
        body,
        grid=(GRID,),
        in_specs=[
            pl.BlockSpec((NCORES, ROWBLK, H1), lambda i: (0, i, 0)),
            pl.BlockSpec((ROWBLK, H1), lambda i: (i, 0)),
            pl.BlockSpec((ROWBLK, 1), lambda i: (i, 0)),
            pl.BlockSpec((1, H1), lambda i: (0, 0)),
            pl.BlockSpec((H1, H2), lambda i: (0, 0)),
        ],
        out_specs=pl.BlockSpec((ROWBLK, H2), lambda i: (i, 0)),
        out_shape=jax.ShapeDtypeStruct((N_PAD, H2), jnp.float32),
    )(acc, hws1, dis, b1, Wc2)


def _tc_post(acc, hws2, dis, b2, Wp, bp):
    def body(acc_ref, hws_ref, dis_ref, b2_ref, Wp_ref, bp_ref, out_ref):
        dis = dis_ref[...]
        agg = acc_ref[0] + acc_ref[1] + hws_ref[...]
        t = _leaky(agg * dis + b2_ref[...])
        t = t / jnp.sqrt(jnp.sum(t * t, axis=1, keepdims=True))
        out_ref[...] = jnp.dot(t, Wp_ref[...], preferred_element_type=jnp.float32) + bp_ref[...]

    return pl.pallas_call(
        body,
        grid=(GRID,),
        in_specs=[
            pl.BlockSpec((NCORES, ROWBLK, H2), lambda i: (0, i, 0)),
            pl.BlockSpec((ROWBLK, H2), lambda i: (i, 0)),
            pl.BlockSpec((ROWBLK, 1), lambda i: (i, 0)),
            pl.BlockSpec((1, H2), lambda i: (0, 0)),
            pl.BlockSpec((H2, 2), lambda i: (0, 0)),
            pl.BlockSpec((1, 2), lambda i: (0, 0)),
        ],
        out_specs=pl.BlockSpec((ROWBLK, 2), lambda i: (i, 0)),
        out_shape=jax.ShapeDtypeStruct((N, 2), jnp.float32),
    )(acc, hws2, dis, b2, Wp, bp)


def kernel(x, edge_index, W_pre1, b_pre1, W_pre2, b_pre2, W_c1, b_c1, W_c2, b_c2,
           W_post, b_post):
    er = edge_index.reshape(2, NW, NBATCH, EB)

    ones_row = jnp.ones((EB, DEGW), jnp.float32)
    z16 = jnp.zeros((N_PAD, DEGW), jnp.float32)
    z64 = jnp.zeros((N_PAD, H1), jnp.float32)
    z32 = jnp.zeros((N_PAD, H2), jnp.float32)
    dcnt = _sc_degree(er, ones_row, z16)
    hws1, dis = _tc_pre(x, dcnt, W_pre1, b_pre1.reshape(1, -1),
                        W_pre2, b_pre2.reshape(1, -1), W_c1)
    acc1 = _sc_conv(hws1, er, z64, H1)
    hws2 = _tc_mid(acc1, hws1, dis, b_c1.reshape(1, -1), W_c2)
    acc2 = _sc_conv(hws2, er, z32, H2)
    out = _tc_post(acc2, hws2, dis, b_c2.reshape(1, -1), W_post, b_post.reshape(1, -1))
    return out


# speedup vs baseline: 2.2701x; 1.0975x over previous
import functools

import jax
import jax.numpy as jnp
from jax import lax
from jax.experimental import pallas as pl
from jax.experimental.pallas import tpu as pltpu
from jax.experimental.pallas import tpu_sc as plsc

N = 10000
E = 320000
D = 128
H1 = 64
H2 = 32

NCORES = 2
NSUB = 16
NW = NCORES * NSUB
EPT = E // NW
EB = 80
NBATCH = EPT // EB
NBUF_DEG = 5
NBUF = 5
NPT = 632
N_PAD = NSUB * NPT
DEGW = 16

ROWBLK = 1000
GRID = N // ROWBLK


def _sc_mesh():
    return plsc.VectorSubcoreMesh(core_axis_name="c", subcore_axis_name="s")


def _load_idx(er_hbm, row, wid, idx_v):
    pltpu.sync_copy(er_hbm.at[row, wid], idx_v)


def _sc_degree(er, ones_row, zeros16):
    @functools.partial(
        pl.kernel,
        out_type=jax.ShapeDtypeStruct((NCORES, N_PAD, DEGW), jnp.float32),
        mesh=_sc_mesh(),
        compiler_params=pltpu.CompilerParams(use_tc_tiling_on_sc=False),
        scratch_types=[
            pltpu.VMEM((NBATCH, EB), jnp.int32),
            pltpu.VMEM((EB, DEGW), jnp.float32),
            pltpu.VMEM_SHARED((N_PAD, DEGW), jnp.float32),
        ] + [pltpu.SemaphoreType.DMA] * NBUF_DEG,
    )
    def k(er_hbm, ones_hbm, zero_hbm, out_hbm, dst_v, ones_v, acc_sh, *sems):
        c = lax.axis_index("c")
        s = lax.axis_index("s")
        wid = s * NCORES + c
        _load_idx(er_hbm, 1, wid, dst_v)
        pltpu.sync_copy(ones_hbm, ones_v)
        pltpu.sync_copy(zero_hbm.at[pl.ds(s * NPT, NPT)],
                        acc_sh.at[pl.ds(s * NPT, NPT)])
        plsc.subcore_barrier()

        def body(k_, carry):
            for i in range(NBUF_DEG):
                j = NBUF_DEG * k_ + i

                @pl.when(k_ > 0)
                def _():
                    pltpu.make_async_copy(
                        ones_v, acc_sh.at[dst_v.at[j - NBUF_DEG]],
                        sems[i]).wait()

                pltpu.async_copy(ones_v, acc_sh.at[dst_v.at[j]], sems[i],
                                 add=True)
            return carry

        lax.fori_loop(0, NBATCH // NBUF_DEG, body, 0)
        for i in range(NBUF_DEG):
            j = NBATCH - NBUF_DEG + i
            pltpu.make_async_copy(ones_v, acc_sh.at[dst_v.at[j]], sems[i]).wait()
        plsc.subcore_barrier()
        pltpu.sync_copy(acc_sh.at[pl.ds(s * NPT, NPT)],
                        out_hbm.at[c, pl.ds(s * NPT, NPT)])

    return k(er, ones_row, zeros16)


def _sc_conv(hws, er, zeros, feat):
    @functools.partial(
        pl.kernel,
        out_type=jax.ShapeDtypeStruct((NCORES, N_PAD, feat), jnp.float32),
        mesh=_sc_mesh(),
        compiler_params=pltpu.CompilerParams(use_tc_tiling_on_sc=False),
        scratch_types=[
            pltpu.VMEM((NBATCH, EB), jnp.int32),
            pltpu.VMEM((NBATCH, EB), jnp.int32),
            pltpu.VMEM_SHARED((N_PAD, feat), jnp.float32),
            pltpu.VMEM_SHARED((N_PAD, feat), jnp.float32),
        ] + [pltpu.VMEM((EB, feat), jnp.float32)] * NBUF
          + [pltpu.SemaphoreType.DMA] * NBUF,
    )
    def k(hws_hbm, er_hbm, zero_hbm, out_hbm, src_v, dst_v, acc_sh, hws_sh,
          *rest):
        bufs = rest[:NBUF]
        gsem = rest[NBUF:]
        c = lax.axis_index("c")
        s = lax.axis_index("s")
        wid = s * NCORES + c
        _load_idx(er_hbm, 0, wid, src_v)
        _load_idx(er_hbm, 1, wid, dst_v)
        pltpu.sync_copy(zero_hbm.at[pl.ds(s * NPT, NPT)],
                        acc_sh.at[pl.ds(s * NPT, NPT)])
        pltpu.sync_copy(hws_hbm.at[pl.ds(s * NPT, NPT)],
                        hws_sh.at[pl.ds(s * NPT, NPT)])
        plsc.subcore_barrier()

        for i in range(NBUF):
            pltpu.async_copy(hws_sh.at[src_v.at[i]], bufs[i], gsem[i])

        def body(k_, carry):
            for i in range(NBUF):
                j = NBUF * k_ + i
                pltpu.make_async_copy(
                    hws_sh.at[src_v.at[j]], bufs[i], gsem[i]).wait()
                pltpu.sync_copy(bufs[i], acc_sh.at[dst_v.at[j]], add=True)

                @pl.when(j + NBUF < NBATCH)
                def _():
                    pltpu.async_copy(
                        hws_sh.at[src_v.at[j + NBUF]], bufs[i], gsem[i])
            return carry

        lax.fori_loop(0, NBATCH // NBUF, body, 0)
        plsc.subcore_barrier()
        pltpu.sync_copy(acc_sh.at[pl.ds(s * NPT, NPT)],
                        out_hbm.at[c, pl.ds(s * NPT, NPT)])

    return k(hws, er, zeros)


def _leaky(x):
    return jnp.where(x >= 0, x, 0.01 * x)


def _tc_pre(x, dcnt, W1, b1, W2, b2, Wc1):
    def body(x_ref, d_ref, W1_ref, b1_ref, W2_ref, b2_ref, Wc1_ref, hws_ref, dis_ref):
        h = jnp.dot(x_ref[...], W1_ref[...], preferred_element_type=jnp.float32)
        h = _leaky(h + b1_ref[...])
        h = jnp.dot(h, W2_ref[...], preferred_element_type=jnp.float32)
        h = _leaky(h + b2_ref[...])
        hw = jnp.dot(h, Wc1_ref[...], preferred_element_type=jnp.float32)
        deg = d_ref[0, :, 0:1] + d_ref[1, :, 0:1] + 1.0
        dis = 1.0 / jnp.sqrt(deg)
        hws_ref[...] = hw * dis
        dis_ref[...] = dis

    return pl.pallas_call(
        body,
        grid=(GRID,),
        in_specs=[
            pl.BlockSpec((ROWBLK, D), lambda i: (i, 0)),
            pl.BlockSpec((NCORES, ROWBLK, DEGW), lambda i: (0, i, 0)),
            pl.BlockSpec((D, 256), lambda i: (0, 0)),
            pl.BlockSpec((1, 256), lambda i: (0, 0)),
            pl.BlockSpec((256, D), lambda i: (0, 0)),
            pl.BlockSpec((1, D), lambda i: (0, 0)),
            pl.BlockSpec((D, H1), lambda i: (0, 0)),
        ],
        out_specs=[
            pl.BlockSpec((ROWBLK, H1), lambda i: (i, 0)),
            pl.BlockSpec((ROWBLK, 1), lambda i: (i, 0)),
        ],
        out_shape=[
            jax.ShapeDtypeStruct((N_PAD, H1), jnp.float32),
            jax.ShapeDtypeStruct((N, 1), jnp.float32),
        ],
    )(x, dcnt, W1, b1, W2, b2, Wc1)


def _tc_mid(acc, hws1, dis, b1, Wc2):
    def body(acc_ref, hws_ref, dis_ref, b1_ref, Wc2_ref, out_ref):
        dis = dis_ref[...]
        agg = acc_ref[0] + acc_ref[1] + hws_ref[...]
        t = _leaky(agg * dis + b1_ref[...])
        t = t / jnp.sqrt(jnp.sum(t * t, axis=1, keepdims=True))
        out_ref[...] = jnp.dot(t, Wc2_ref[...], preferred_element_type=jnp.float32) * dis

    return pl.pallas_call(
        body,
        grid=(GRID,),
        in_specs=[
            pl.BlockSpec((NCORES, ROWBLK, H1), lambda i: (0, i, 0)),
            pl.BlockSpec((ROWBLK, H1), lambda i: (i, 0)),
            pl.BlockSpec((ROWBLK, 1), lambda i: (i, 0)),
            pl.BlockSpec((1, H1), lambda i: (0, 0)),
            pl.BlockSpec((H1, H2), lambda i: (0, 0)),
        ],
        out_specs=pl.BlockSpec((ROWBLK, H2), lambda i: (i, 0)),
        out_shape=jax.ShapeDtypeStruct((N_PAD, H2), jnp.float32),
    )(acc, hws1, dis, b1, Wc2)


def _tc_post(acc, hws2, dis, b2, Wp, bp):
    def body(acc_ref, hws_ref, dis_ref, b2_ref, Wp_ref, bp_ref, out_ref):
        dis = dis_ref[...]
        agg = acc_ref[0] + acc_ref[1] + hws_ref[...]
        t = _leaky(agg * dis + b2_ref[...])
        t = t / jnp.sqrt(jnp.sum(t * t, axis=1, keepdims=True))
        out_ref[...] = jnp.dot(t, Wp_ref[...], preferred_element_type=jnp.float32) + bp_ref[...]

    return pl.pallas_call(
        body,
        grid=(GRID,),
        in_specs=[
            pl.BlockSpec((NCORES, ROWBLK, H2), lambda i: (0, i, 0)),
            pl.BlockSpec((ROWBLK, H2), lambda i: (i, 0)),
            pl.BlockSpec((ROWBLK, 1), lambda i: (i, 0)),
            pl.BlockSpec((1, H2), lambda i: (0, 0)),
            pl.BlockSpec((H2, 2), lambda i: (0, 0)),
            pl.BlockSpec((1, 2), lambda i: (0, 0)),
        ],
        out_specs=pl.BlockSpec((ROWBLK, 2), lambda i: (i, 0)),
        out_shape=jax.ShapeDtypeStruct((N, 2), jnp.float32),
    )(acc, hws2, dis, b2, Wp, bp)


def kernel(x, edge_index, W_pre1, b_pre1, W_pre2, b_pre2, W_c1, b_c1, W_c2, b_c2,
           W_post, b_post):
    er = edge_index.reshape(2, NW, NBATCH, EB)

    ones_row = jnp.ones((EB, DEGW), jnp.float32)
    z16 = jnp.zeros((N_PAD, DEGW), jnp.float32)
    z64 = jnp.zeros((N_PAD, H1), jnp.float32)
    z32 = jnp.zeros((N_PAD, H2), jnp.float32)
    dcnt = _sc_degree(er, ones_row, z16)
    hws1, dis = _tc_pre(x, dcnt, W_pre1, b_pre1.reshape(1, -1),
                        W_pre2, b_pre2.reshape(1, -1), W_c1)
    acc1 = _sc_conv(hws1, er, z64, H1)
    hws2 = _tc_mid(acc1, hws1, dis, b_c1.reshape(1, -1), W_c2)
    acc2 = _sc_conv(hws2, er, z32, H2)
    out = _tc_post(acc2, hws2, dis, b_c2.reshape(1, -1), W_post, b_post.reshape(1, -1))
    return out


# speedup vs baseline: 2.6151x; 1.1520x over previous
import functools

import jax
import jax.numpy as jnp
from jax import lax
from jax.experimental import pallas as pl
from jax.experimental.pallas import tpu as pltpu
from jax.experimental.pallas import tpu_sc as plsc

N = 10000
E = 320000
D = 128
H1 = 64
H2 = 32

NCORES = 2
NSUB = 16
NW = NCORES * NSUB
EPT = E // NW
EB = 80
NBATCH = EPT // EB
NBUF_DEG = 5
NBUF = 5
NPT = 632
N_PAD = NSUB * NPT
DEGW = 16

ROWBLK = 1000
GRID = N // ROWBLK


def _sc_mesh():
    return plsc.VectorSubcoreMesh(core_axis_name="c", subcore_axis_name="s")


def _load_idx(er_hbm, row, wid, idx_v):
    pltpu.sync_copy(er_hbm.at[row, wid], idx_v)


def _sc_degree(er, ones_row, zeros16):
    @functools.partial(
        pl.kernel,
        out_type=jax.ShapeDtypeStruct((NCORES, N_PAD, DEGW), jnp.float32),
        mesh=_sc_mesh(),
        compiler_params=pltpu.CompilerParams(use_tc_tiling_on_sc=False),
        scratch_types=[
            pltpu.VMEM((NBATCH, EB), jnp.int32),
            pltpu.VMEM((EB, DEGW), jnp.float32),
            pltpu.VMEM_SHARED((N_PAD, DEGW), jnp.float32),
        ] + [pltpu.SemaphoreType.DMA] * NBUF_DEG,
    )
    def k(er_hbm, ones_hbm, zero_hbm, out_hbm, dst_v, ones_v, acc_sh, *sems):
        c = lax.axis_index("c")
        s = lax.axis_index("s")
        wid = s * NCORES + c
        _load_idx(er_hbm, 1, wid, dst_v)
        pltpu.sync_copy(ones_hbm, ones_v)
        pltpu.sync_copy(zero_hbm.at[pl.ds(s * NPT, NPT)],
                        acc_sh.at[pl.ds(s * NPT, NPT)])
        plsc.subcore_barrier()

        def body(k_, carry):
            for i in range(NBUF_DEG):
                j = NBUF_DEG * k_ + i

                @pl.when(k_ > 0)
                def _():
                    pltpu.make_async_copy(
                        ones_v, acc_sh.at[dst_v.at[j - NBUF_DEG]],
                        sems[i]).wait()

                pltpu.async_copy(ones_v, acc_sh.at[dst_v.at[j]], sems[i],
                                 add=True)
            return carry

        lax.fori_loop(0, NBATCH // NBUF_DEG, body, 0)
        for i in range(NBUF_DEG):
            j = NBATCH - NBUF_DEG + i
            pltpu.make_async_copy(ones_v, acc_sh.at[dst_v.at[j]], sems[i]).wait()
        plsc.subcore_barrier()
        pltpu.sync_copy(acc_sh.at[pl.ds(s * NPT, NPT)],
                        out_hbm.at[c, pl.ds(s * NPT, NPT)])

    return k(er, ones_row, zeros16)


def _sc_conv(hws, er, zeros, feat):
    @functools.partial(
        pl.kernel,
        out_type=jax.ShapeDtypeStruct((NCORES, N_PAD, feat), jnp.float32),
        mesh=_sc_mesh(),
        compiler_params=pltpu.CompilerParams(use_tc_tiling_on_sc=False),
        scratch_types=[
            pltpu.VMEM((NBATCH, EB), jnp.int32),
            pltpu.VMEM((NBATCH, EB), jnp.int32),
            pltpu.VMEM_SHARED((N_PAD, feat), jnp.float32),
        ] + [pltpu.VMEM((EB, feat), jnp.float32)] * NBUF
          + [pltpu.SemaphoreType.DMA] * NBUF,
    )
    def k(hws_hbm, er_hbm, zero_hbm, out_hbm, src_v, dst_v, acc_sh, *rest):
        bufs = rest[:NBUF]
        gsem = rest[NBUF:]
        c = lax.axis_index("c")
        s = lax.axis_index("s")
        wid = s * NCORES + c
        _load_idx(er_hbm, 0, wid, src_v)
        _load_idx(er_hbm, 1, wid, dst_v)
        pltpu.sync_copy(zero_hbm.at[pl.ds(s * NPT, NPT)],
                        acc_sh.at[pl.ds(s * NPT, NPT)])
        plsc.subcore_barrier()

        for i in range(NBUF):
            pltpu.async_copy(hws_hbm.at[src_v.at[i]], bufs[i], gsem[i])

        def body(k_, carry):
            for i in range(NBUF):
                j = NBUF * k_ + i
                pltpu.make_async_copy(
                    hws_hbm.at[src_v.at[j]], bufs[i], gsem[i]).wait()
                pltpu.sync_copy(bufs[i], acc_sh.at[dst_v.at[j]], add=True)

                @pl.when(j + NBUF < NBATCH)
                def _():
                    pltpu.async_copy(
                        hws_hbm.at[src_v.at[j + NBUF]], bufs[i], gsem[i])
            return carry

        lax.fori_loop(0, NBATCH // NBUF, body, 0)
        plsc.subcore_barrier()
        pltpu.sync_copy(acc_sh.at[pl.ds(s * NPT, NPT)],
                        out_hbm.at[c, pl.ds(s * NPT, NPT)])

    return k(hws, er, zeros)


def _leaky(x):
    return jnp.where(x >= 0, x, 0.01 * x)


def _tc_pre(x, dcnt, W1, b1, W2, b2, Wc1):
    def body(x_ref, d_ref, W1_ref, b1_ref, W2_ref, b2_ref, Wc1_ref, hws_ref, dis_ref):
        h = jnp.dot(x_ref[...], W1_ref[...], preferred_element_type=jnp.float32)
        h = _leaky(h + b1_ref[...])
        h = jnp.dot(h, W2_ref[...], preferred_element_type=jnp.float32)
        h = _leaky(h + b2_ref[...])
        hw = jnp.dot(h, Wc1_ref[...], preferred_element_type=jnp.float32)
        deg = d_ref[0, :, 0:1] + d_ref[1, :, 0:1] + 1.0
        dis = 1.0 / jnp.sqrt(deg)
        hws_ref[...] = hw * dis
        dis_ref[...] = dis

    return pl.pallas_call(
        body,
        grid=(GRID,),
        in_specs=[
            pl.BlockSpec((ROWBLK, D), lambda i: (i, 0)),
            pl.BlockSpec((NCORES, ROWBLK, DEGW), lambda i: (0, i, 0)),
            pl.BlockSpec((D, 256), lambda i: (0, 0)),
            pl.BlockSpec((1, 256), lambda i: (0, 0)),
            pl.BlockSpec((256, D), lambda i: (0, 0)),
            pl.BlockSpec((1, D), lambda i: (0, 0)),
            pl.BlockSpec((D, H1), lambda i: (0, 0)),
        ],
        out_specs=[
            pl.BlockSpec((ROWBLK, H1), lambda i: (i, 0)),
            pl.BlockSpec((ROWBLK, 1), lambda i: (i, 0)),
        ],
        out_shape=[
            jax.ShapeDtypeStruct((N_PAD, H1), jnp.float32),
            jax.ShapeDtypeStruct((N, 1), jnp.float32),
        ],
    )(x, dcnt, W1, b1, W2, b2, Wc1)


def _tc_mid(acc, hws1, dis, b1, Wc2):
    def body(acc_ref, hws_ref, dis_ref, b1_ref, Wc2_ref, out_ref):
        dis = dis_ref[...]
        agg = acc_ref[0] + acc_ref[1] + hws_ref[...]
        t = _leaky(agg * dis + b1_ref[...])
        t = t / jnp.sqrt(jnp.sum(t * t, axis=1, keepdims=True))
        out_ref[...] = jnp.dot(t, Wc2_ref[...], preferred_element_type=jnp.float32) * dis

    return pl.pallas_call(
        body,
        grid=(GRID,),
        in_specs=[
            pl.BlockSpec((NCORES, ROWBLK, H1), lambda i: (0, i, 0)),
            pl.BlockSpec((ROWBLK, H1), lambda i: (i, 0)),
            pl.BlockSpec((ROWBLK, 1), lambda i: (i, 0)),
            pl.BlockSpec((1, H1), lambda i: (0, 0)),
            pl.BlockSpec((H1, H2), lambda i: (0, 0)),
        ],
        out_specs=pl.BlockSpec((ROWBLK, H2), lambda i: (i, 0)),
        out_shape=jax.ShapeDtypeStruct((N_PAD, H2), jnp.float32),
    )(acc, hws1, dis, b1, Wc2)


def _tc_post(acc, hws2, dis, b2, Wp, bp):
    def body(acc_ref, hws_ref, dis_ref, b2_ref, Wp_ref, bp_ref, out_ref):
        dis = dis_ref[...]
        agg = acc_ref[0] + acc_ref[1] + hws_ref[...]
        t = _leaky(agg * dis + b2_ref[...])
        t = t / jnp.sqrt(jnp.sum(t * t, axis=1, keepdims=True))
        out_ref[...] = jnp.dot(t, Wp_ref[...], preferred_element_type=jnp.float32) + bp_ref[...]

    return pl.pallas_call(
        body,
        grid=(GRID,),
        in_specs=[
            pl.BlockSpec((NCORES, ROWBLK, H2), lambda i: (0, i, 0)),
            pl.BlockSpec((ROWBLK, H2), lambda i: (i, 0)),
            pl.BlockSpec((ROWBLK, 1), lambda i: (i, 0)),
            pl.BlockSpec((1, H2), lambda i: (0, 0)),
            pl.BlockSpec((H2, 2), lambda i: (0, 0)),
            pl.BlockSpec((1, 2), lambda i: (0, 0)),
        ],
        out_specs=pl.BlockSpec((ROWBLK, 2), lambda i: (i, 0)),
        out_shape=jax.ShapeDtypeStruct((N, 2), jnp.float32),
    )(acc, hws2, dis, b2, Wp, bp)


def kernel(x, edge_index, W_pre1, b_pre1, W_pre2, b_pre2, W_c1, b_c1, W_c2, b_c2,
           W_post, b_post):
    er = edge_index.reshape(2, NW, NBATCH, EB)

    ones_row = jnp.ones((EB, DEGW), jnp.float32)
    z16 = jnp.zeros((N_PAD, DEGW), jnp.float32)
    z64 = jnp.zeros((N_PAD, H1), jnp.float32)
    z32 = jnp.zeros((N_PAD, H2), jnp.float32)
    dcnt = _sc_degree(er, ones_row, z16)
    hws1, dis = _tc_pre(x, dcnt, W_pre1, b_pre1.reshape(1, -1),
                        W_pre2, b_pre2.reshape(1, -1), W_c1)
    acc1 = _sc_conv(hws1, er, z64, H1)
    hws2 = _tc_mid(acc1, hws1, dis, b_c1.reshape(1, -1), W_c2)
    acc2 = _sc_conv(hws2, er, z32, H2)
    out = _tc_post(acc2, hws2, dis, b_c2.reshape(1, -1), W_post, b_post.reshape(1, -1))
    return out


# speedup vs baseline: 2.6452x; 1.0115x over previous
import functools

import jax
import jax.numpy as jnp
from jax import lax
from jax.experimental import pallas as pl
from jax.experimental.pallas import tpu as pltpu
from jax.experimental.pallas import tpu_sc as plsc

N = 10000
E = 320000
D = 128
H1 = 64
H2 = 32

NCORES = 2
NSUB = 16
NW = NCORES * NSUB
EPT = E // NW
EB = 80
NBATCH = EPT // EB
NBUF_DEG = 5
NBUF = 10
NTAIL = NBATCH - (NBATCH // NBUF) * NBUF
NPT = 632
N_PAD = NSUB * NPT
DEGW = 16

ROWBLK = 1000
GRID = N // ROWBLK


def _sc_mesh():
    return plsc.VectorSubcoreMesh(core_axis_name="c", subcore_axis_name="s")


def _load_idx(er_hbm, row, wid, idx_v):
    pltpu.sync_copy(er_hbm.at[row, wid], idx_v)


def _sc_degree(er, ones_row, zeros16):
    @functools.partial(
        pl.kernel,
        out_type=jax.ShapeDtypeStruct((NCORES, N_PAD, DEGW), jnp.float32),
        mesh=_sc_mesh(),
        compiler_params=pltpu.CompilerParams(use_tc_tiling_on_sc=False),
        scratch_types=[
            pltpu.VMEM((NBATCH, EB), jnp.int32),
            pltpu.VMEM((EB, DEGW), jnp.float32),
            pltpu.VMEM_SHARED((N_PAD, DEGW), jnp.float32),
        ] + [pltpu.SemaphoreType.DMA] * NBUF_DEG,
    )
    def k(er_hbm, ones_hbm, zero_hbm, out_hbm, dst_v, ones_v, acc_sh, *sems):
        c = lax.axis_index("c")
        s = lax.axis_index("s")
        wid = s * NCORES + c
        _load_idx(er_hbm, 1, wid, dst_v)
        pltpu.sync_copy(ones_hbm, ones_v)
        pltpu.sync_copy(zero_hbm.at[pl.ds(s * NPT, NPT)],
                        acc_sh.at[pl.ds(s * NPT, NPT)])
        plsc.subcore_barrier()

        def body(k_, carry):
            for i in range(NBUF_DEG):
                j = NBUF_DEG * k_ + i

                @pl.when(k_ > 0)
                def _():
                    pltpu.make_async_copy(
                        ones_v, acc_sh.at[dst_v.at[j - NBUF_DEG]],
                        sems[i]).wait()

                pltpu.async_copy(ones_v, acc_sh.at[dst_v.at[j]], sems[i],
                                 add=True)
            return carry

        lax.fori_loop(0, NBATCH // NBUF_DEG, body, 0)
        for i in range(NBUF_DEG):
            j = NBATCH - NBUF_DEG + i
            pltpu.make_async_copy(ones_v, acc_sh.at[dst_v.at[j]], sems[i]).wait()
        plsc.subcore_barrier()
        pltpu.sync_copy(acc_sh.at[pl.ds(s * NPT, NPT)],
                        out_hbm.at[c, pl.ds(s * NPT, NPT)])

    return k(er, ones_row, zeros16)


def _sc_conv(hws, er, zeros, feat):
    @functools.partial(
        pl.kernel,
        out_type=jax.ShapeDtypeStruct((NCORES, N_PAD, feat), jnp.float32),
        mesh=_sc_mesh(),
        compiler_params=pltpu.CompilerParams(use_tc_tiling_on_sc=False),
        scratch_types=[
            pltpu.VMEM((NBATCH, EB), jnp.int32),
            pltpu.VMEM((NBATCH, EB), jnp.int32),
            pltpu.VMEM_SHARED((N_PAD, feat), jnp.float32),
        ] + [pltpu.VMEM((EB, feat), jnp.float32)] * NBUF
          + [pltpu.SemaphoreType.DMA] * NBUF,
    )
    def k(hws_hbm, er_hbm, zero_hbm, out_hbm, src_v, dst_v, acc_sh, *rest):
        bufs = rest[:NBUF]
        gsem = rest[NBUF:]
        c = lax.axis_index("c")
        s = lax.axis_index("s")
        wid = s * NCORES + c
        _load_idx(er_hbm, 0, wid, src_v)
        _load_idx(er_hbm, 1, wid, dst_v)
        pltpu.sync_copy(zero_hbm.at[pl.ds(s * NPT, NPT)],
                        acc_sh.at[pl.ds(s * NPT, NPT)])
        plsc.subcore_barrier()

        for i in range(NBUF):
            pltpu.async_copy(hws_hbm.at[src_v.at[i]], bufs[i], gsem[i])

        def body(k_, carry):
            for i in range(NBUF):
                j = NBUF * k_ + i
                pltpu.make_async_copy(
                    hws_hbm.at[src_v.at[j]], bufs[i], gsem[i]).wait()
                pltpu.sync_copy(bufs[i], acc_sh.at[dst_v.at[j]], add=True)

                @pl.when(j + NBUF < NBATCH)
                def _():
                    pltpu.async_copy(
                        hws_hbm.at[src_v.at[j + NBUF]], bufs[i], gsem[i])
            return carry

        lax.fori_loop(0, NBATCH // NBUF, body, 0)
        for i in range(NTAIL):
            j = (NBATCH // NBUF) * NBUF + i
            pltpu.make_async_copy(
                hws_hbm.at[src_v.at[j]], bufs[i], gsem[i]).wait()
            pltpu.sync_copy(bufs[i], acc_sh.at[dst_v.at[j]], add=True)
        plsc.subcore_barrier()
        pltpu.sync_copy(acc_sh.at[pl.ds(s * NPT, NPT)],
                        out_hbm.at[c, pl.ds(s * NPT, NPT)])

    return k(hws, er, zeros)


def _leaky(x):
    return jnp.where(x >= 0, x, 0.01 * x)


def _tc_pre(x, dcnt, W1, b1, W2, b2, Wc1):
    def body(x_ref, d_ref, W1_ref, b1_ref, W2_ref, b2_ref, Wc1_ref, hws_ref, dis_ref):
        h = jnp.dot(x_ref[...], W1_ref[...], preferred_element_type=jnp.float32)
        h = _leaky(h + b1_ref[...])
        h = jnp.dot(h, W2_ref[...], preferred_element_type=jnp.float32)
        h = _leaky(h + b2_ref[...])
        hw = jnp.dot(h, Wc1_ref[...], preferred_element_type=jnp.float32)
        deg = d_ref[0, :, 0:1] + d_ref[1, :, 0:1] + 1.0
        dis = 1.0 / jnp.sqrt(deg)
        hws_ref[...] = hw * dis
        dis_ref[...] = dis

    return pl.pallas_call(
        body,
        grid=(GRID,),
        in_specs=[
            pl.BlockSpec((ROWBLK, D), lambda i: (i, 0)),
            pl.BlockSpec((NCORES, ROWBLK, DEGW), lambda i: (0, i, 0)),
            pl.BlockSpec((D, 256), lambda i: (0, 0)),
            pl.BlockSpec((1, 256), lambda i: (0, 0)),
            pl.BlockSpec((256, D), lambda i: (0, 0)),
            pl.BlockSpec((1, D), lambda i: (0, 0)),
            pl.BlockSpec((D, H1), lambda i: (0, 0)),
        ],
        out_specs=[
            pl.BlockSpec((ROWBLK, H1), lambda i: (i, 0)),
            pl.BlockSpec((ROWBLK, 1), lambda i: (i, 0)),
        ],
        out_shape=[
            jax.ShapeDtypeStruct((N_PAD, H1), jnp.float32),
            jax.ShapeDtypeStruct((N, 1), jnp.float32),
        ],
    )(x, dcnt, W1, b1, W2, b2, Wc1)


def _tc_mid(acc, hws1, dis, b1, Wc2):
    def body(acc_ref, hws_ref, dis_ref, b1_ref, Wc2_ref, out_ref):
        dis = dis_ref[...]
        agg = acc_ref[0] + acc_ref[1] + hws_ref[...]
        t = _leaky(agg * dis + b1_ref[...])
        t = t / jnp.sqrt(jnp.sum(t * t, axis=1, keepdims=True))
        out_ref[...] = jnp.dot(t, Wc2_ref[...], preferred_element_type=jnp.float32) * dis

    return pl.pallas_call(
        body,
        grid=(GRID,),
        in_specs=[
            pl.BlockSpec((NCORES, ROWBLK, H1), lambda i: (0, i, 0)),
            pl.BlockSpec((ROWBLK, H1), lambda i: (i, 0)),
            pl.BlockSpec((ROWBLK, 1), lambda i: (i, 0)),
            pl.BlockSpec((1, H1), lambda i: (0, 0)),
            pl.BlockSpec((H1, H2), lambda i: (0, 0)),
        ],
        out_specs=pl.BlockSpec((ROWBLK, H2), lambda i: (i, 0)),
        out_shape=jax.ShapeDtypeStruct((N_PAD, H2), jnp.float32),
    )(acc, hws1, dis, b1, Wc2)


def _tc_post(acc, hws2, dis, b2, Wp, bp):
    def body(acc_ref, hws_ref, dis_ref, b2_ref, Wp_ref, bp_ref, out_ref):
        dis = dis_ref[...]
        agg = acc_ref[0] + acc_ref[1] + hws_ref[...]
        t = _leaky(agg * dis + b2_ref[...])
        t = t / jnp.sqrt(jnp.sum(t * t, axis=1, keepdims=True))
        out_ref[...] = jnp.dot(t, Wp_ref[...], preferred_element_type=jnp.float32) + bp_ref[...]

    return pl.pallas_call(
        body,
        grid=(GRID,),
        in_specs=[
            pl.BlockSpec((NCORES, ROWBLK, H2), lambda i: (0, i, 0)),
            pl.BlockSpec((ROWBLK, H2), lambda i: (i, 0)),
            pl.BlockSpec((ROWBLK, 1), lambda i: (i, 0)),
            pl.BlockSpec((1, H2), lambda i: (0, 0)),
            pl.BlockSpec((H2, 2), lambda i: (0, 0)),
            pl.BlockSpec((1, 2), lambda i: (0, 0)),
        ],
        out_specs=pl.BlockSpec((ROWBLK, 2), lambda i: (i, 0)),
        out_shape=jax.ShapeDtypeStruct((N, 2), jnp.float32),
    )(acc, hws2, dis, b2, Wp, bp)


def kernel(x, edge_index, W_pre1, b_pre1, W_pre2, b_pre2, W_c1, b_c1, W_c2, b_c2,
           W_post, b_post):
    er = edge_index.reshape(2, NW, NBATCH, EB)

    ones_row = jnp.ones((EB, DEGW), jnp.float32)
    z16 = jnp.zeros((N_PAD, DEGW), jnp.float32)
    z64 = jnp.zeros((N_PAD, H1), jnp.float32)
    z32 = jnp.zeros((N_PAD, H2), jnp.float32)
    dcnt = _sc_degree(er, ones_row, z16)
    hws1, dis = _tc_pre(x, dcnt, W_pre1, b_pre1.reshape(1, -1),
                        W_pre2, b_pre2.reshape(1, -1), W_c1)
    acc1 = _sc_conv(hws1, er, z64, H1)
    hws2 = _tc_mid(acc1, hws1, dis, b_c1.reshape(1, -1), W_c2)
    acc2 = _sc_conv(hws2, er, z32, H2)
    out = _tc_post(acc2, hws2, dis, b_c2.reshape(1, -1), W_post, b_post.reshape(1, -1))
    return out
